# Initial kernel scaffold; baseline (speedup 1.0000x reference)
import jax
import jax.numpy as jnp
from jax.experimental import pallas as pl


def kernel(x, center_mean, center_pre_variance, center_constant, context_mean, context_pre_variance, context_constant):
    raise NotImplementedError("write your pallas kernel here")


# speedup vs baseline: 2.5933x; 2.5933x over previous
import functools

import jax
import jax.numpy as jnp
from jax import lax
from jax.experimental import pallas as pl
from jax.experimental.pallas import tpu as pltpu
from jax.experimental.pallas import tpu_sc as plsc

V = 100001
EMB = 128
DP = 2 * EMB
DS = 3 * EMB
EPS = 1e-23
TWO_PI = 6.283185307179586

NC, NS = 2, 16
NW = NC * NS
NBUF = 4
FIRE_AHEAD = 2


def _prep_body(m_ref, pv_ref, c_ref, out_ref, k_ref):
    m = m_ref[:]
    x = pv_ref[:]
    v = jnp.maximum(x, 0.0) + jnp.log1p(jnp.exp(-jnp.abs(x)))
    p = 1.0 / v
    h = m * p
    out_ref[:] = jnp.concatenate([p, h], axis=1)
    k_ref[:] = c_ref[:] - 0.5 * jnp.sum(m * h, axis=-1, keepdims=True)


def _prep(mean, pre_var, const):
    rb = 1024
    grid = pl.cdiv(V, rb)
    packed, k = pl.pallas_call(
        _prep_body,
        grid=(grid,),
        in_specs=[
            pl.BlockSpec((rb, EMB), lambda i: (i, 0)),
            pl.BlockSpec((rb, EMB), lambda i: (i, 0)),
            pl.BlockSpec((rb, 1), lambda i: (i, 0)),
        ],
        out_specs=[
            pl.BlockSpec((rb, DP), lambda i: (i, 0)),
            pl.BlockSpec((rb, 1), lambda i: (i, 0)),
        ],
        out_shape=(
            jax.ShapeDtypeStruct((V, DP), jnp.float32),
            jax.ShapeDtypeStruct((V, 1), jnp.float32),
        ),
    )(mean, pre_var, const)
    return packed, jnp.reshape(k, (V,))


def _sc_body(bpw, xg_hbm, cen_hbm, ctx_hbm, kcen_hbm, kctx_hbm,
             rows_out, sums_out, xv, *scr):
    cbufs = scr[0:NBUF]
    xbufs = scr[NBUF:2 * NBUF]
    kcbufs = scr[2 * NBUF:3 * NBUF]
    kxbufs = scr[3 * NBUF:4 * NBUF]
    stags = scr[4 * NBUF:5 * NBUF]
    gsc = scr[5 * NBUF:6 * NBUF]
    gsx = scr[6 * NBUF:7 * NBUF]
    gskc = scr[7 * NBUF:8 * NBUF]
    gskx = scr[8 * NBUF:9 * NBUF]
    osr = scr[9 * NBUF:10 * NBUF]
    oss = scr[10 * NBUF:11 * NBUF]

    wid = lax.axis_index("s") * NC + lax.axis_index("c")
    base = wid * bpw
    pltpu.sync_copy(xg_hbm.at[pl.ds(base, bpw)], xv)

    def gdesc(s, b):
        ic = xv.at[s, pl.ds(0, 24)]
        ix = xv.at[s, pl.ds(24, 24)]
        return (
            pltpu.make_async_copy(cen_hbm.at[ic], cbufs[b], gsc[b]),
            pltpu.make_async_copy(ctx_hbm.at[ix], xbufs[b], gsx[b]),
            pltpu.make_async_copy(kcen_hbm.at[ic], kcbufs[b].at[pl.ds(0, 24)], gskc[b]),
            pltpu.make_async_copy(kctx_hbm.at[ix], kxbufs[b].at[pl.ds(0, 24)], gskx[b]),
        )

    def odesc(s, b):
        return (
            pltpu.make_async_copy(cbufs[b], rows_out.at[base + s], osr[b]),
            pltpu.make_async_copy(stags[b], sums_out.at[base + s], oss[b]),
        )

    for s in range(FIRE_AHEAD):
        for d in gdesc(s, s % NBUF):
            d.start()

    lanes = lax.iota(jnp.int32, 16)

    @pl.loop(0, bpw // NBUF)
    def _grp(g):
        s0 = g * NBUF
        for b in range(NBUF):
            s = s0 + b
            fb = (b + FIRE_AHEAD) % NBUF
            for d in gdesc(s, b):
                d.wait()
            dr, dsm = odesc(s, b)
            dr.start()
            accs = [xbufs[b][0, pl.ds(16 * d, 16)] for d in range(DP // 16)]
            for r in range(1, 20):
                for d in range(DP // 16):
                    accs[d] = accs[d] + xbufs[b][r, pl.ds(16 * d, 16)]
            for d in range(DP // 16):
                stags[b][pl.ds(16 * d, 16)] = accs[d]
            kx0 = kxbufs[b][pl.ds(0, 16)]
            kx1 = jnp.where(lanes < 4, kxbufs[b][pl.ds(16, 16)], 0.0)
            zero16 = jnp.zeros((16,), jnp.float32)
            stags[b][pl.ds(256, 16)] = kx0 + kx1
            stags[b][pl.ds(272, 16)] = kcbufs[b][pl.ds(0, 16)]
            stags[b][pl.ds(288, 16)] = jnp.where(
                lanes < 5, kcbufs[b][pl.ds(16, 16)], 0.0)
            for d in range(19, DS // 16):
                stags[b][pl.ds(16 * d, 16)] = zero16
            dsm.start()
            nxt = s + FIRE_AHEAD

            @pl.when(nxt < bpw)
            def _fire():
                @pl.when(nxt >= NBUF)
                def _drain():
                    for d in odesc(nxt - NBUF, fb):
                        d.wait()
                for d in gdesc(nxt, fb):
                    d.start()

    for s in range(bpw - NBUF, bpw):
        for d in odesc(s, s % NBUF):
            d.wait()


def _sc_gather(xg, cen_packed, ctx_packed, kcen, kctx):
    b = xg.shape[0]
    bpw = b // NW
    mesh = plsc.VectorSubcoreMesh(
        core_axis_name="c", subcore_axis_name="s", num_cores=NC, num_subcores=NS)
    scratch = (
        [pltpu.VMEM((bpw, 48), jnp.int32)]
        + [pltpu.VMEM((24, DP), jnp.float32) for _ in range(NBUF)]
        + [pltpu.VMEM((24, DP), jnp.float32) for _ in range(NBUF)]
        + [pltpu.VMEM((32,), jnp.float32) for _ in range(NBUF)]
        + [pltpu.VMEM((32,), jnp.float32) for _ in range(NBUF)]
        + [pltpu.VMEM((DS,), jnp.float32) for _ in range(NBUF)]
        + [pltpu.SemaphoreType.DMA for _ in range(6 * NBUF)]
    )
    fn = pl.kernel(
        functools.partial(_sc_body, bpw),
        out_type=(
            jax.ShapeDtypeStruct((b, 24, DP), jnp.float32),
            jax.ShapeDtypeStruct((b, DS), jnp.float32),
        ),
        mesh=mesh,
        scratch_types=scratch,
    )
    return fn(xg, cen_packed, ctx_packed, kcen, kctx)


def _score_body(rows_ref, sums_ref, pos_ref, neg_ref):
    sums = sums_ref[:]
    p_sum = sums[:, 0:EMB]
    h_sum = sums[:, EMB:2 * EMB]
    k_sum = jnp.sum(sums[:, 2 * EMB:2 * EMB + 16], axis=-1, keepdims=True)
    kj = sums[:, 272:293]
    rows = rows_ref[:, 0:21, :]
    a = rows[:, :, 0:EMB] + p_sum[:, None, :]
    bv = rows[:, :, EMB:2 * EMB] + h_sum[:, None, :]
    r = 1.0 / a
    t = 0.5 * jnp.log(TWO_PI * r + EPS) + (0.5 * bv * bv) * r
    sc = jnp.sum(t, axis=-1) + kj + k_sum
    pos_ref[:] = sc[:, 0:1][:, :, None]
    neg_ref[:] = sc[:, 1:21][:, :, None]


def _score(rows, sums):
    b = rows.shape[0]
    bb = 128
    return pl.pallas_call(
        _score_body,
        grid=(b // bb,),
        in_specs=[
            pl.BlockSpec((bb, 24, DP), lambda i: (i, 0, 0)),
            pl.BlockSpec((bb, DS), lambda i: (i, 0)),
        ],
        out_specs=[
            pl.BlockSpec((bb, 1, 1), lambda i: (i, 0, 0)),
            pl.BlockSpec((bb, 20, 1), lambda i: (i, 0, 0)),
        ],
        out_shape=(
            jax.ShapeDtypeStruct((b, 1, 1), jnp.float32),
            jax.ShapeDtypeStruct((b, 20, 1), jnp.float32),
        ),
    )(rows, sums)


def kernel(x, center_mean, center_pre_variance, center_constant,
           context_mean, context_pre_variance, context_constant):
    b = x.shape[0]
    x = (x + V) % V
    zeros3 = jnp.zeros((b, 3), jnp.int32)
    zeros4 = jnp.zeros((b, 4), jnp.int32)
    xg = jnp.concatenate([x[:, :21], zeros3, x[:, 21:], zeros4], axis=1)

    cen_packed, kcen = _prep(center_mean, center_pre_variance, center_constant)
    ctx_packed, kctx = _prep(context_mean, context_pre_variance, context_constant)
    rows, sums = _sc_gather(xg, cen_packed, ctx_packed, kcen, kctx)
    return _score(rows, sums)
